# Initial kernel scaffold; baseline (speedup 1.0000x reference)
import functools

import jax
import jax.numpy as jnp
from jax import lax
from jax.experimental import pallas as pl
from jax.experimental.pallas import tpu as pltpu
from jax.experimental.pallas import tpu_sc as plsc

N = 10000
E = 320000
NPAD = 10048
RPT = NPAD // 16
NCHA = 79
NCHB = 157
F32 = jnp.float32


def _sc_agg(nch, w, with_deg):
    mesh = plsc.VectorSubcoreMesh(core_axis_name="c", subcore_axis_name="s")

    def body(x_hbm, src_hbm, dst_hbm, *rest):
        if with_deg:
            out_hbm, deg_hbm = rest[0], rest[1]
            rest = rest[2:]
        else:
            out_hbm = rest[0]
            rest = rest[1:]
        src_v, dst_v, rows_v, zeros_v, ones_v, zd_v, acc, accd, sem = rest
        c = lax.axis_index("c")
        s = lax.axis_index("s")
        wid = c * 16 + s
        base = s * RPT

        wc = w // 16

        def zinit(i, carry):
            zeros_v[i // wc, pl.ds((i % wc) * 16, 16)] = jnp.zeros((16,), F32)
            return carry

        lax.fori_loop(0, 64 * wc, zinit, 0)

        def oinit(i, carry):
            ones_v[i, :] = jnp.ones((16,), F32)
            zd_v[i % 64, :] = jnp.zeros((16,), F32)
            return carry

        lax.fori_loop(0, 128, oinit, 0)

        for k in range(9):
            pltpu.sync_copy(zeros_v, acc.at[pl.ds(base + k * 64, 64)])
        pltpu.sync_copy(zeros_v.at[pl.ds(0, 52)], acc.at[pl.ds(base + 576, 52)])
        if with_deg:
            for k in range(9):
                pltpu.sync_copy(zd_v, accd.at[pl.ds(base + k * 64, 64)])
            pltpu.sync_copy(zd_v.at[pl.ds(0, 52)], accd.at[pl.ds(base + 576, 52)])
        plsc.subcore_barrier()

        pltpu.sync_copy(src_hbm.at[wid], src_v)
        pltpu.sync_copy(dst_hbm.at[wid], dst_v)

        def step(j, carry):
            pltpu.async_copy(x_hbm.at[src_v.at[j]], rows_v, sem).wait()
            pltpu.sync_copy(rows_v, acc.at[dst_v.at[j]], add=True)
            if with_deg:
                pltpu.sync_copy(ones_v, accd.at[dst_v.at[j]], add=True)
            return carry

        lax.fori_loop(0, nch, step, 0)
        plsc.subcore_barrier()

        pltpu.sync_copy(acc.at[pl.ds(base, RPT)], out_hbm.at[c, pl.ds(base, RPT)])
        if with_deg:
            pltpu.sync_copy(accd.at[pl.ds(base, RPT)], deg_hbm.at[c, pl.ds(base, RPT)])

    out_type = [jax.ShapeDtypeStruct((2, NPAD, w), F32)]
    if with_deg:
        out_type.append(jax.ShapeDtypeStruct((2, NPAD, 16), F32))
    scratch = [
        pltpu.VMEM((nch, 128), jnp.int32),
        pltpu.VMEM((nch, 128), jnp.int32),
        pltpu.VMEM((128, w), F32),
        pltpu.VMEM((64, w), F32),
        pltpu.VMEM((128, 16), F32),
        pltpu.VMEM((64, 16), F32),
        pltpu.VMEM_SHARED((NPAD, w), F32),
        pltpu.VMEM_SHARED((NPAD, 16), F32),
        pltpu.SemaphoreType.DMA,
    ]
    return pl.kernel(body, out_type=tuple(out_type), mesh=mesh,
                     scratch_types=scratch)


BM = 1000


def _recip_deg(d_ref):
    dg = d_ref[0, :, :1] + d_ref[1, :, :1]
    return 1.0 / jnp.maximum(dg, 1.0)


def _tc1_body(x_ref, a_ref, d_ref, wl_ref, bl_ref, wr_ref, o_ref):
    r = _recip_deg(d_ref)
    an = (a_ref[0] + a_ref[1]) * r
    h = (jnp.dot(an, wl_ref[...], preferred_element_type=F32) + bl_ref[...]
         + jnp.dot(x_ref[...], wr_ref[...], preferred_element_type=F32))
    h = jnp.maximum(h, 0.0)
    o_ref[0] = h[:, :128]
    o_ref[1] = h[:, 128:]


def _tc2_body(a_ref, d_ref, h_ref, wl_ref, bl_ref, wr_ref, wc_ref, bc_ref,
              p_ref, q_ref):
    r = _recip_deg(d_ref)
    h2 = (jnp.dot(a_ref[0] * r, wl_ref[:128, :], preferred_element_type=F32)
          + jnp.dot(a_ref[1] * r, wl_ref[128:, :], preferred_element_type=F32)
          + bl_ref[...]
          + jnp.dot(h_ref[0], wr_ref[:128, :], preferred_element_type=F32)
          + jnp.dot(h_ref[1], wr_ref[128:, :], preferred_element_type=F32))
    h2 = jnp.maximum(h2, 0.0)
    pq = jnp.dot(h2, wc_ref[...], preferred_element_type=F32) + bc_ref[...]
    p_ref[...] = pq[:, :64]
    q_ref[...] = pq[:, 64:]


def _tc3_body(a_ref, d_ref, q_ref, o_ref):
    r = _recip_deg(d_ref)
    o_ref[...] = (a_ref[0] + a_ref[1]) * r + q_ref[...]


def _tc1(x, agg1, deg, Wl1, bl1, Wr1):
    return pl.pallas_call(
        _tc1_body,
        grid=(N // BM,),
        in_specs=[
            pl.BlockSpec((BM, 128), lambda i: (i, 0)),
            pl.BlockSpec((2, BM, 128), lambda i: (0, i, 0)),
            pl.BlockSpec((2, BM, 16), lambda i: (0, i, 0)),
            pl.BlockSpec((128, 256), lambda i: (0, 0)),
            pl.BlockSpec((1, 256), lambda i: (0, 0)),
            pl.BlockSpec((128, 256), lambda i: (0, 0)),
        ],
        out_specs=pl.BlockSpec((2, BM, 128), lambda i: (0, i, 0)),
        out_shape=jax.ShapeDtypeStruct((2, N, 128), F32),
    )(x, agg1, deg, Wl1, bl1, Wr1)


def _tc2(agg2, deg, h1s, Wl2, bl2, Wr2, Wcat, bcat):
    return pl.pallas_call(
        _tc2_body,
        grid=(N // BM,),
        in_specs=[
            pl.BlockSpec((2, BM, 128), lambda i: (0, i, 0)),
            pl.BlockSpec((2, BM, 16), lambda i: (0, i, 0)),
            pl.BlockSpec((2, BM, 128), lambda i: (0, i, 0)),
            pl.BlockSpec((256, 256), lambda i: (0, 0)),
            pl.BlockSpec((1, 256), lambda i: (0, 0)),
            pl.BlockSpec((256, 256), lambda i: (0, 0)),
            pl.BlockSpec((256, 128), lambda i: (0, 0)),
            pl.BlockSpec((1, 128), lambda i: (0, 0)),
        ],
        out_specs=[
            pl.BlockSpec((BM, 64), lambda i: (i, 0)),
            pl.BlockSpec((BM, 64), lambda i: (i, 0)),
        ],
        out_shape=[
            jax.ShapeDtypeStruct((N, 64), F32),
            jax.ShapeDtypeStruct((N, 64), F32),
        ],
    )(agg2, deg, h1s, Wl2, bl2, Wr2, Wcat, bcat)


def _tc3(agg3, deg, q):
    return pl.pallas_call(
        _tc3_body,
        grid=(N // BM,),
        in_specs=[
            pl.BlockSpec((2, BM, 64), lambda i: (0, i, 0)),
            pl.BlockSpec((2, BM, 16), lambda i: (0, i, 0)),
            pl.BlockSpec((BM, 64), lambda i: (i, 0)),
        ],
        out_specs=pl.BlockSpec((BM, 64), lambda i: (i, 0)),
        out_shape=jax.ShapeDtypeStruct((N, 64), F32),
    )(agg3, deg, q)


def _edge_layout(src, dst, nch, tiles):
    pad = tiles * nch * 128 - E
    s = jnp.concatenate([src, jnp.zeros((pad,), jnp.int32)]).reshape(tiles, nch, 128)
    d = jnp.concatenate([dst, jnp.full((pad,), N, jnp.int32)]).reshape(tiles, nch, 128)
    return s, d


def kernel(x, edge_index, Wl1, bl1, Wr1, Wl2, bl2, Wr2, Wl3, bl3, Wr3):
    src = edge_index[0]
    dst = edge_index[1]

    srcA, dstA = _edge_layout(src, dst, NCHA, 32)
    srcBh, dstBh = _edge_layout(src, dst, NCHB, 16)
    srcB = jnp.concatenate([srcBh[None], srcBh[None] + N], axis=0).reshape(32, NCHB, 128)
    dstB = jnp.concatenate([dstBh[None], dstBh[None]], axis=0).reshape(32, NCHB, 128)

    agg1, deg = _sc_agg(NCHA, 128, True)(x, srcA, dstA)
    h1s = _tc1(x, agg1, deg, Wl1, bl1.reshape(1, -1), Wr1)

    agg2 = _sc_agg(NCHB, 128, False)(h1s.reshape(2 * N, 128), srcB, dstB)
    Wcat = jnp.concatenate([Wl3, Wr3], axis=1)
    bcat = jnp.concatenate([jnp.zeros((64,), F32), bl3]).reshape(1, 128)
    p, q = _tc2(agg2, deg, h1s, Wl2, bl2.reshape(1, -1), Wr2, Wcat, bcat)

    agg3 = _sc_agg(NCHA, 64, False)(p, srcA, dstA)
    return _tc3(agg3, deg, q)


# speedup vs baseline: 3.3343x; 3.3343x over previous
import jax
import jax.numpy as jnp
from jax import lax
from jax.experimental import pallas as pl
from jax.experimental.pallas import tpu as pltpu
from jax.experimental.pallas import tpu_sc as plsc

N = 10000
E = 320000
HALF = 5000
ACCN = 5120
DUMMY = 5056
RPT = ACCN // 16
NCH = 157
EPAD = 16 * NCH * 128
F32 = jnp.float32


def _sc_agg(npass, deg_pass):
    mesh = plsc.VectorSubcoreMesh(core_axis_name="c", subcore_axis_name="s")

    def body(x_hbm, src_hbm, dst_hbm, out_hbm, src_v, dst_v, rows_v, zeros_v,
             acc, sem):
        c = lax.axis_index("c")
        s = lax.axis_index("s")
        wid = c * 16 + s
        base = s * RPT

        def zinit(i, carry):
            zeros_v[i // 8, pl.ds((i % 8) * 16, 16)] = jnp.zeros((16,), F32)
            return carry

        lax.fori_loop(0, 64 * 8, zinit, 0)

        pltpu.sync_copy(dst_hbm.at[wid], dst_v)

        def zero_acc():
            for blk in range(5):
                pltpu.sync_copy(zeros_v, acc.at[pl.ds(base + blk * 64, 64)])

        def copy_out(plane):
            pltpu.sync_copy(acc.at[pl.ds(base, RPT)],
                            out_hbm.at[plane, pl.ds(base, RPT)])

        for k in range(npass):
            zero_acc()
            plsc.subcore_barrier()
            pltpu.sync_copy(src_hbm.at[k * 32 + wid], src_v)

            def step(j, carry):
                pltpu.async_copy(x_hbm.at[src_v.at[j]], rows_v, sem).wait()
                pltpu.sync_copy(rows_v, acc.at[dst_v.at[j]], add=True)
                return carry

            lax.fori_loop(0, NCH, step, 0)
            plsc.subcore_barrier()
            copy_out(k * 2 + c)

        if deg_pass:
            def oinit(i, carry):
                rows_v[i // 8, pl.ds((i % 8) * 16, 16)] = jnp.ones((16,), F32)
                return carry

            lax.fori_loop(0, 128 * 8, oinit, 0)
            zero_acc()
            plsc.subcore_barrier()

            def dstep(j, carry):
                pltpu.sync_copy(rows_v, acc.at[dst_v.at[j]], add=True)
                return carry

            lax.fori_loop(0, NCH, dstep, 0)
            plsc.subcore_barrier()
            copy_out(npass * 2 + c)

    nplanes = (npass + (1 if deg_pass else 0)) * 2
    return pl.kernel(
        body,
        out_type=jax.ShapeDtypeStruct((nplanes, ACCN, 128), F32),
        mesh=mesh,
        scratch_types=[
            pltpu.VMEM((NCH, 128), jnp.int32),
            pltpu.VMEM((NCH, 128), jnp.int32),
            pltpu.VMEM((128, 128), F32),
            pltpu.VMEM((64, 128), F32),
            pltpu.VMEM_SHARED((ACCN, 128), F32),
            pltpu.SemaphoreType.DMA,
        ],
    )


BM = 1000

def _split(p):
    return lambda i: (2 * p + i // 5, i % 5, 0)


_DENSE2 = lambda i: (i, 0)
_FULL3 = lambda i: (0, i, 0)
_WHOLE = lambda i: (0, 0)


def _recip_deg(d_ref):
    return 1.0 / jnp.maximum(d_ref[0, :, :1], 1.0)


def _tc1_body(x_ref, a_ref, d_ref, wl_ref, bl_ref, wr_ref, o_ref):
    an = a_ref[0] * _recip_deg(d_ref)
    h = (jnp.dot(an, wl_ref[...], preferred_element_type=F32) + bl_ref[...]
         + jnp.dot(x_ref[...], wr_ref[...], preferred_element_type=F32))
    h = jnp.maximum(h, 0.0)
    o_ref[0] = h[:, :128]
    o_ref[1] = h[:, 128:]


def _tc2_body(a0_ref, a1_ref, d_ref, h_ref, wl_ref, bl_ref, wr_ref, wc_ref,
              bc_ref, o_ref):
    r = _recip_deg(d_ref)
    h2 = (jnp.dot(a0_ref[0] * r, wl_ref[:128, :], preferred_element_type=F32)
          + jnp.dot(a1_ref[0] * r, wl_ref[128:, :], preferred_element_type=F32)
          + bl_ref[...]
          + jnp.dot(h_ref[0], wr_ref[:128, :], preferred_element_type=F32)
          + jnp.dot(h_ref[1], wr_ref[128:, :], preferred_element_type=F32))
    h2 = jnp.maximum(h2, 0.0)
    o_ref[...] = jnp.dot(h2, wc_ref[...], preferred_element_type=F32) + bc_ref[...]


def _tc3_body(a_ref, d_ref, pq_ref, o_ref):
    o_ref[...] = a_ref[0, :, :64] * _recip_deg(d_ref) + pq_ref[:, 64:]


def _tc1(x, aggd1, Wl1, bl1, Wr1):
    return pl.pallas_call(
        _tc1_body,
        grid=(N // BM,),
        in_specs=[
            pl.BlockSpec((BM, 128), _DENSE2),
            pl.BlockSpec((1, BM, 128), _split(0)),
            pl.BlockSpec((1, BM, 128), _split(1)),
            pl.BlockSpec((128, 256), _WHOLE),
            pl.BlockSpec((1, 256), _WHOLE),
            pl.BlockSpec((128, 256), _WHOLE),
        ],
        out_specs=pl.BlockSpec((2, BM, 128), _FULL3),
        out_shape=jax.ShapeDtypeStruct((2, N, 128), F32),
    )(x, aggd1, aggd1, Wl1, bl1, Wr1)


def _tc2(agg2, aggd1, h1s, Wl2, bl2, Wr2, Wcat, bcat):
    return pl.pallas_call(
        _tc2_body,
        grid=(N // BM,),
        in_specs=[
            pl.BlockSpec((1, BM, 128), _split(0)),
            pl.BlockSpec((1, BM, 128), _split(1)),
            pl.BlockSpec((1, BM, 128), _split(1)),
            pl.BlockSpec((2, BM, 128), _FULL3),
            pl.BlockSpec((256, 256), _WHOLE),
            pl.BlockSpec((1, 256), _WHOLE),
            pl.BlockSpec((256, 256), _WHOLE),
            pl.BlockSpec((256, 128), _WHOLE),
            pl.BlockSpec((1, 128), _WHOLE),
        ],
        out_specs=pl.BlockSpec((BM, 128), _DENSE2),
        out_shape=jax.ShapeDtypeStruct((N, 128), F32),
    )(agg2, agg2, aggd1, h1s, Wl2, bl2, Wr2, Wcat, bcat)


def _tc3(agg3, aggd1, pq):
    return pl.pallas_call(
        _tc3_body,
        grid=(N // BM,),
        in_specs=[
            pl.BlockSpec((1, BM, 128), _split(0)),
            pl.BlockSpec((1, BM, 128), _split(1)),
            pl.BlockSpec((BM, 128), _DENSE2),
        ],
        out_specs=pl.BlockSpec((BM, 64), _DENSE2),
        out_shape=jax.ShapeDtypeStruct((N, 64), F32),
    )(agg3, aggd1, pq)


def kernel(x, edge_index, Wl1, bl1, Wr1, Wl2, bl2, Wr2, Wl3, bl3, Wr3):
    src = edge_index[0]
    dst = edge_index[1]

    pad = EPAD - E
    srcP = jnp.concatenate([src, jnp.zeros((pad,), jnp.int32)])
    dstP = jnp.concatenate([dst, jnp.full((pad,), -1, jnp.int32)])
    dl = []
    for cc in (0, 1):
        lo = cc * HALF
        inr = (dstP >= lo) & (dstP < lo + HALF)
        dl.append(jnp.where(inr, dstP - lo, DUMMY).reshape(1, 16, NCH, 128))
    dstL = jnp.concatenate(dl, axis=0).reshape(32, NCH, 128)
    srcT = srcP.reshape(1, 16, NCH, 128)
    srcS = jnp.concatenate([srcT, srcT], axis=0).reshape(32, NCH, 128)
    srcS2 = jnp.concatenate([srcS, srcS + N], axis=0)

    aggd1 = _sc_agg(1, True)(x, srcS, dstL)
    h1s = _tc1(x, aggd1, Wl1, bl1.reshape(1, -1), Wr1)

    agg2 = _sc_agg(2, False)(h1s.reshape(2 * N, 128), srcS2, dstL)

    Wcat = jnp.concatenate([Wl3, Wr3], axis=1)
    bcat = jnp.concatenate([jnp.zeros((64,), F32), bl3]).reshape(1, 128)
    pq = _tc2(agg2, aggd1, h1s, Wl2, bl2.reshape(1, -1), Wr2, Wcat, bcat)

    agg3 = _sc_agg(1, False)(pq, srcS, dstL)
    return _tc3(agg3, aggd1, pq)
